# baseline (speedup 1.0000x reference)
import functools

import jax
import jax.numpy as jnp
from jax.experimental import pallas as pl
from jax.experimental.pallas import tpu as pltpu

N_NODES = 10000
N_EDGES = 160000
HIDDEN = 256


def _mm_body(x_ref, w_ref, b_ref, o_ref):
    o_ref[...] = (
        jnp.dot(x_ref[...], w_ref[...], preferred_element_type=jnp.float32)
        + b_ref[...]
    )


@functools.partial(jax.jit, static_argnames=("bm",))
def _matmul_bias(x, w, b, bm=1024):
    m, k = x.shape
    n = w.shape[1]
    grid = (pl.cdiv(m, bm),)
    return pl.pallas_call(
        _mm_body,
        grid=grid,
        in_specs=[
            pl.BlockSpec((bm, k), lambda i: (i, 0)),
            pl.BlockSpec((k, n), lambda i: (0, 0)),
            pl.BlockSpec((1, n), lambda i: (0, 0)),
        ],
        out_specs=pl.BlockSpec((bm, n), lambda i: (i, 0)),
        out_shape=jax.ShapeDtypeStruct((m, n), jnp.float32),
    )(x, w, b.reshape(1, n))


def _bn(x, g, b):
    mu = jnp.mean(x, axis=0, keepdims=True)
    var = jnp.var(x, axis=0, keepdims=True)
    return g * (x - mu) / jnp.sqrt(var + 1e-5) + b


def kernel(dataset_idx, edge_index, h, e, training_flag, params):
    src = edge_index[0]
    dst = edge_index[1]
    h = _matmul_bias(h, params["emb_h_W"], params["emb_h_b"])
    e = _matmul_bias(e, params["emb_e_W"], params["emb_e_b"])
    for lp in params["layers"]:
        h_in, e_in = h, e
        w_cat = jnp.concatenate(
            [lp["A_W"], lp["B_W"], lp["D_W"], lp["E_W"]], axis=1
        )
        b_cat = jnp.concatenate(
            [lp["A_b"], lp["B_b"], lp["D_b"], lp["E_b"]], axis=0
        )
        proj = _matmul_bias(h, w_cat, b_cat)
        Ah = proj[:, :HIDDEN]
        Bh = proj[:, HIDDEN : 2 * HIDDEN]
        Dh = proj[:, 2 * HIDDEN : 3 * HIDDEN]
        Eh = proj[:, 3 * HIDDEN :]
        Ce = _matmul_bias(e, lp["C_W"], lp["C_b"])
        e_new = Ce + Dh[src] + Eh[dst]
        sigma = jax.nn.sigmoid(e_new)
        num = jax.ops.segment_sum(sigma * Bh[src], dst, num_segments=N_NODES)
        den = jax.ops.segment_sum(sigma, dst, num_segments=N_NODES)
        h_new = Ah + num / (den + 1e-6)
        h_new = jax.nn.relu(_bn(h_new, lp["bn_h_g"], lp["bn_h_b"]))
        e_new = jax.nn.relu(_bn(e_new, lp["bn_e_g"], lp["bn_e_b"]))
        h = h_in + h_new
        e = e_in + e_new
    hg = jnp.concatenate([jnp.sum(h, axis=0), jnp.sum(e, axis=0)])
    x = hg
    n = len(params["mlp_Ws"])
    for i in range(n):
        x = x @ params["mlp_Ws"][i] + params["mlp_bs"][i]
        if i < n - 1:
            x = jax.nn.relu(x)
    return x
